# baseline (speedup 1.0000x reference)
import jax
import jax.numpy as jnp
from jax import lax
from jax.experimental import pallas as pl
from jax.experimental.pallas import tpu as pltpu

N_POINTS = 100000
HIDDEN = 128
N_EIGEN = 5
BLOCK = 12800
BF = jnp.bfloat16
F32 = jnp.float32

_DN_T = (((0,), (0,)), ((), ()))


def _mlp_kernel(t_ref, w1_ref, b1_ref, w2_ref, b2_ref, w3_ref, b3_ref, o_ref,
                h1s_ref):
    t = t_ref[...]
    t_hi = t.astype(BF)
    t_lo = (t - t_hi.astype(F32)).astype(BF)
    ones = jnp.ones_like(t_hi)
    x = jnp.concatenate([t_hi, t_lo, t_hi, ones, ones], axis=0)

    w1 = w1_ref[...]
    b1 = b1_ref[...]
    w1hi = w1.astype(BF)
    w1lo = (w1 - w1hi.astype(F32)).astype(BF)
    b1hi = b1.astype(BF)
    b1lo = (b1 - b1hi.astype(F32)).astype(BF)
    a1t = jnp.concatenate([w1hi, w1hi, w1lo, b1hi, b1lo], axis=0)

    h = lax.dot_general(a1t, x, _DN_T, preferred_element_type=F32)
    h1s_ref[0:HIDDEN, :] = jnp.maximum(h.astype(BF), jnp.array(0, BF))
    h1s_ref[HIDDEN:HIDDEN + 1, :] = ones
    w2aug = jnp.concatenate(
        [w2_ref[...].astype(BF), b2_ref[...].astype(BF)], axis=0)
    h = lax.dot_general(w2aug, h1s_ref[...], _DN_T, preferred_element_type=F32)
    h = jnp.maximum(h.astype(BF), jnp.array(0, BF))
    o = lax.dot_general(w3_ref[...].astype(BF), h, _DN_T, preferred_element_type=F32)
    o_ref[...] = o + b3_ref[...].reshape(N_EIGEN, 1)


def kernel(t_values, W1, b1, W2, b2, W3, b3):
    t_row = t_values.reshape(1, N_POINTS)
    b1r = b1.reshape(1, HIDDEN)
    b2r = b2.reshape(1, HIDDEN)
    b3r = b3.reshape(1, N_EIGEN)

    grid = (pl.cdiv(N_POINTS, BLOCK),)
    rep = lambda shape: pl.BlockSpec(shape, lambda i: (0, 0))
    out_t = pl.pallas_call(
        _mlp_kernel,
        grid=grid,
        in_specs=[
            pl.BlockSpec((1, BLOCK), lambda i: (0, i)),
            rep((1, HIDDEN)),
            rep((1, HIDDEN)),
            rep((HIDDEN, HIDDEN)),
            rep((1, HIDDEN)),
            rep((HIDDEN, N_EIGEN)),
            rep((1, N_EIGEN)),
        ],
        out_specs=pl.BlockSpec((N_EIGEN, BLOCK), lambda i: (0, i)),
        out_shape=jax.ShapeDtypeStruct((N_EIGEN, N_POINTS), jnp.float32),
        scratch_shapes=[pltpu.VMEM((HIDDEN + 1, BLOCK), BF)],
        compiler_params=pltpu.CompilerParams(
            dimension_semantics=("parallel",),
        ),
    )(t_row, W1, b1r, W2, b2r, W3, b3r)
    return out_t.T


# speedup vs baseline: 1.0461x; 1.0461x over previous
import jax
import jax.numpy as jnp
from jax import lax
from jax.experimental import pallas as pl
from jax.experimental.pallas import tpu as pltpu

N_POINTS = 100000
HIDDEN = 128
N_EIGEN = 5
BLOCK = 51200
BF = jnp.bfloat16
F32 = jnp.float32

_DN_T = (((0,), (0,)), ((), ()))


def _mlp_kernel(t_ref, w1_ref, b1_ref, w2_ref, b2_ref, w3_ref, b3_ref, o_ref,
                h1s_ref):
    t = t_ref[...]
    t_hi = t.astype(BF)
    t_lo = (t - t_hi.astype(F32)).astype(BF)
    ones = jnp.ones_like(t_hi)
    x = jnp.concatenate([t_hi, t_lo, t_hi, ones, ones], axis=0)

    w1 = w1_ref[...]
    b1 = b1_ref[...]
    w1hi = w1.astype(BF)
    w1lo = (w1 - w1hi.astype(F32)).astype(BF)
    b1hi = b1.astype(BF)
    b1lo = (b1 - b1hi.astype(F32)).astype(BF)
    a1t = jnp.concatenate([w1hi, w1hi, w1lo, b1hi, b1lo], axis=0)

    h = lax.dot_general(a1t, x, _DN_T, preferred_element_type=F32)
    h1s_ref[0:HIDDEN, :] = jnp.maximum(h.astype(BF), jnp.array(0, BF))
    h1s_ref[HIDDEN:HIDDEN + 1, :] = ones
    w2aug = jnp.concatenate(
        [w2_ref[...].astype(BF), b2_ref[...].astype(BF)], axis=0)
    h = lax.dot_general(w2aug, h1s_ref[...], _DN_T, preferred_element_type=F32)
    h = jnp.maximum(h.astype(BF), jnp.array(0, BF))
    o = lax.dot_general(w3_ref[...].astype(BF), h, _DN_T, preferred_element_type=F32)
    o_ref[...] = o + b3_ref[...].reshape(N_EIGEN, 1)


def kernel(t_values, W1, b1, W2, b2, W3, b3):
    t_row = t_values.reshape(1, N_POINTS)
    b1r = b1.reshape(1, HIDDEN)
    b2r = b2.reshape(1, HIDDEN)
    b3r = b3.reshape(1, N_EIGEN)

    grid = (pl.cdiv(N_POINTS, BLOCK),)
    rep = lambda shape: pl.BlockSpec(shape, lambda i: (0, 0))
    out_t = pl.pallas_call(
        _mlp_kernel,
        grid=grid,
        in_specs=[
            pl.BlockSpec((1, BLOCK), lambda i: (0, i)),
            rep((1, HIDDEN)),
            rep((1, HIDDEN)),
            rep((HIDDEN, HIDDEN)),
            rep((1, HIDDEN)),
            rep((HIDDEN, N_EIGEN)),
            rep((1, N_EIGEN)),
        ],
        out_specs=pl.BlockSpec((N_EIGEN, BLOCK), lambda i: (0, i)),
        out_shape=jax.ShapeDtypeStruct((N_EIGEN, N_POINTS), jnp.float32),
        scratch_shapes=[pltpu.VMEM((HIDDEN + 1, BLOCK), BF)],
        compiler_params=pltpu.CompilerParams(
            dimension_semantics=("arbitrary",),
            fuse_transposed_lhs_in_matmul=True,
        ),
    )(t_row, W1, b1r, W2, b2r, W3, b3r)
    return out_t.T
